# Initial kernel scaffold; baseline (speedup 1.0000x reference)
import jax
import jax.numpy as jnp
from jax.experimental import pallas as pl


def kernel(input):
    raise NotImplementedError("write your pallas kernel here")


# speedup vs baseline: 1.4444x; 1.4444x over previous
import jax
import jax.numpy as jnp
from jax.experimental import pallas as pl

_N_ATOMS = 256
_N_ANGLES = 254
_BATCH = 16384
_CB = 2048


def _body(x_ref, o_ref):
    x = x_ref[...]
    a = x[0:_N_ANGLES]
    b = x[1:_N_ANGLES + 1]
    c = x[2:_N_ANGLES + 2]
    v1 = a - b
    v2 = c - b
    dot = jnp.sum(v1 * v2, axis=1)
    n1 = jnp.sum(v1 * v1, axis=1)
    n2 = jnp.sum(v2 * v2, axis=1)
    o_ref[...] = dot * jax.lax.rsqrt(n1 * n2)


def kernel(input):
    return pl.pallas_call(
        _body,
        grid=(_BATCH // _CB,),
        in_specs=[pl.BlockSpec((_N_ATOMS, 3, _CB), lambda i: (0, 0, i))],
        out_specs=pl.BlockSpec((_N_ANGLES, _CB), lambda i: (0, i)),
        out_shape=jax.ShapeDtypeStruct((_N_ANGLES, _BATCH), jnp.float32),
    )(input)
